# baseline (speedup 1.0000x reference)
import functools

import jax
import jax.numpy as jnp
from jax import lax
from jax.experimental import pallas as pl
from jax.experimental.pallas import tpu as pltpu
from jax.experimental.pallas import tpu_sc as plsc

_NUM_CORES = 2
_NUM_SUBCORES = 16
_NW = _NUM_CORES * _NUM_SUBCORES

_EMB = 128
_PROJ = 384
_OUT_D = _PROJ + 3 * _EMB

_SB = 256
_NBUF = 3


def _sc_gather_body(tag_ids, pretag_ids, pred_ids,
                    tag_tab, pretag_tab, pred_tab,
                    out, idx_tag, idx_pred, idx_pretag,
                    pred_shared, bufs, gsems, wsems, *, chunk):
    wid = lax.axis_index("s") * _NUM_CORES + lax.axis_index("c")
    base = wid * chunk
    n_sb = chunk // _SB

    @pl.when(lax.axis_index("s") == 0)
    def _():
        pltpu.sync_copy(pred_tab, pred_shared)

    pltpu.sync_copy(tag_ids.at[pl.ds(base, chunk)], idx_tag)
    pltpu.sync_copy(pred_ids.at[pl.ds(base, chunk)], idx_pred)
    pltpu.sync_copy(pretag_ids.at[pl.ds(base, chunk)], idx_pretag)
    plsc.subcore_barrier()

    steps = []
    for idx_v, tab, col in (
        (idx_tag, tag_tab, _PROJ),
        (idx_pred, pred_shared, _PROJ + _EMB),
        (idx_pretag, pretag_tab, _PROJ + 2 * _EMB),
    ):
        for j in range(n_sb):
            steps.append((idx_v, tab, col, j))
    n_steps = len(steps)

    def start_gather(s):
        idx_v, tab, _, j = steps[s]
        b = s % _NBUF
        return pltpu.async_copy(
            tab.at[idx_v.at[pl.ds(j * _SB, _SB)]], bufs[b], gsems[b])

    def start_write(s):
        _, _, col, j = steps[s]
        b = s % _NBUF
        return pltpu.async_copy(
            bufs[b], out.at[pl.ds(base + j * _SB, _SB), pl.ds(col, _EMB)],
            wsems[b])

    ghandles, whandles = {}, {}
    for s in range(min(_NBUF - 1, n_steps)):
        ghandles[s] = start_gather(s)
    for s in range(n_steps):
        g = s + _NBUF - 1
        if g < n_steps:
            prev_w = g - _NBUF
            if prev_w >= 0:
                whandles.pop(prev_w).wait()
            ghandles[g] = start_gather(g)
        ghandles.pop(s).wait()
        whandles[s] = start_write(s)
    for s in sorted(whandles):
        whandles.pop(s).wait()


def _sc_gather(tag_ids, pretag_ids, pred_ids, tag_tab, pretag_tab, pred_tab):
    n = tag_ids.shape[0]
    chunk = n // _NW
    mesh = plsc.VectorSubcoreMesh(core_axis_name="c", subcore_axis_name="s")
    return pl.kernel(
        functools.partial(_sc_gather_body, chunk=chunk),
        out_type=jax.ShapeDtypeStruct((n, _OUT_D), jnp.float32),
        mesh=mesh,
        scratch_types=[
            pltpu.VMEM((chunk,), jnp.int32),
            pltpu.VMEM((chunk,), jnp.int32),
            pltpu.VMEM((chunk,), jnp.int32),
            pltpu.VMEM_SHARED((2, _EMB), jnp.float32),
            [pltpu.VMEM((_SB, _EMB), jnp.float32) for _ in range(_NBUF)],
            [pltpu.SemaphoreType.DMA for _ in range(_NBUF)],
            [pltpu.SemaphoreType.DMA for _ in range(_NBUF)],
        ],
    )(tag_ids, pretag_ids, pred_ids, tag_tab, pretag_tab, pred_tab)


def _tc_matmul_body(x_ref, w_ref, b_ref, gath_ref, o_ref):
    del gath_ref
    acc = lax.dot_general(x_ref[...].astype(jnp.bfloat16),
                          w_ref[...].astype(jnp.bfloat16),
                          (((1,), (1,)), ((), ())),
                          preferred_element_type=jnp.float32)
    o_ref[...] = acc + b_ref[...]


def _tc_matmul(x2d, w, b, gath):
    n, d_in = x2d.shape
    tm = 512
    return pl.pallas_call(
        _tc_matmul_body,
        grid=(n // tm,),
        in_specs=[
            pl.BlockSpec((tm, d_in), lambda i: (i, 0)),
            pl.BlockSpec((_PROJ, d_in), lambda i: (0, 0)),
            pl.BlockSpec((1, _PROJ), lambda i: (0, 0)),
            pl.BlockSpec(memory_space=pl.ANY),
        ],
        out_specs=pl.BlockSpec((tm, _PROJ), lambda i: (i, 0)),
        out_shape=jax.ShapeDtypeStruct((n, _OUT_D), jnp.float32),
        input_output_aliases={3: 0},
    )(x2d, w, b.reshape(1, _PROJ), gath)


def kernel(input_layer, tag_ids, pretag_ids, predicate_mask,
           predicate_embeddings, tag_embeddings, pretag_embeddings,
           linear_w, linear_b):
    b, s, d_in = input_layer.shape
    n = b * s
    x2d = input_layer.reshape(n, d_in)
    gath = _sc_gather(
        tag_ids.reshape(n).astype(jnp.int32),
        pretag_ids.reshape(n).astype(jnp.int32),
        predicate_mask.reshape(n).astype(jnp.int32),
        tag_embeddings, pretag_embeddings, predicate_embeddings,
    )
    out = _tc_matmul(x2d, linear_w, linear_b, gath)
    return out.reshape(b, s, _OUT_D)


# speedup vs baseline: 1.1898x; 1.1898x over previous
import functools

import jax
import jax.numpy as jnp
from jax import lax
from jax.experimental import pallas as pl
from jax.experimental.pallas import tpu as pltpu
from jax.experimental.pallas import tpu_sc as plsc

_NUM_CORES = 2
_NUM_SUBCORES = 16
_NW = _NUM_CORES * _NUM_SUBCORES

_EMB = 128
_PROJ = 384
_OUT_D = _PROJ + 3 * _EMB

_SB = 256
_NBUF = 3


def _sc_gather_body(tag_ids, pretag_ids, pred_ids,
                    tag_tab, pretag_tab, pred_tab,
                    out, idx_tag, idx_pred, idx_pretag,
                    pred_shared, bufs, gsems, wsems, *, chunk):
    wid = lax.axis_index("s") * _NUM_CORES + lax.axis_index("c")
    base = wid * chunk
    n_sb = chunk // _SB

    @pl.when(lax.axis_index("s") == 0)
    def _():
        pltpu.sync_copy(pred_tab, pred_shared)

    pltpu.sync_copy(tag_ids.at[pl.ds(base, chunk)], idx_tag)
    pltpu.sync_copy(pred_ids.at[pl.ds(base, chunk)], idx_pred)
    pltpu.sync_copy(pretag_ids.at[pl.ds(base, chunk)], idx_pretag)
    plsc.subcore_barrier()

    steps = []
    for idx_v, tab, col in (
        (idx_tag, tag_tab, _PROJ),
        (idx_pred, pred_shared, _PROJ + _EMB),
        (idx_pretag, pretag_tab, _PROJ + 2 * _EMB),
    ):
        for j in range(n_sb):
            steps.append((idx_v, tab, col, j))
    n_steps = len(steps)

    def start_gather(s):
        idx_v, tab, _, j = steps[s]
        b = s % _NBUF
        return pltpu.async_copy(
            tab.at[idx_v.at[pl.ds(j * _SB, _SB)]], bufs[b], gsems[b])

    def start_write(s):
        _, _, col, j = steps[s]
        b = s % _NBUF
        return pltpu.async_copy(
            bufs[b], out.at[pl.ds(base + j * _SB, _SB), pl.ds(col, _EMB)],
            wsems[b])

    ghandles, whandles = {}, {}
    for s in range(min(_NBUF - 1, n_steps)):
        ghandles[s] = start_gather(s)
    for s in range(n_steps):
        g = s + _NBUF - 1
        if g < n_steps:
            prev_w = g - _NBUF
            if prev_w >= 0:
                whandles.pop(prev_w).wait()
            ghandles[g] = start_gather(g)
        ghandles.pop(s).wait()
        whandles[s] = start_write(s)
    for s in sorted(whandles):
        whandles.pop(s).wait()


def _sc_gather(tag_ids, pretag_ids, pred_ids, tag_tab, pretag_tab, pred_tab):
    n = tag_ids.shape[0]
    chunk = n // _NW
    mesh = plsc.VectorSubcoreMesh(core_axis_name="c", subcore_axis_name="s")
    return pl.kernel(
        functools.partial(_sc_gather_body, chunk=chunk),
        out_type=jax.ShapeDtypeStruct((n, _OUT_D), jnp.float32),
        mesh=mesh,
        scratch_types=[
            pltpu.VMEM((chunk,), jnp.int32),
            pltpu.VMEM((chunk,), jnp.int32),
            pltpu.VMEM((chunk,), jnp.int32),
            pltpu.VMEM_SHARED((2, _EMB), jnp.float32),
            [pltpu.VMEM((_SB, _EMB), jnp.float32) for _ in range(_NBUF)],
            [pltpu.SemaphoreType.DMA for _ in range(_NBUF)],
            [pltpu.SemaphoreType.DMA for _ in range(_NBUF)],
        ],
    )(tag_ids, pretag_ids, pred_ids, tag_tab, pretag_tab, pred_tab)


def _tc_matmul_body(x_ref, w_ref, b_ref, gath_ref, o_ref):
    del gath_ref
    acc = lax.dot_general(x_ref[...].astype(jnp.bfloat16),
                          w_ref[...].astype(jnp.bfloat16),
                          (((1,), (1,)), ((), ())),
                          preferred_element_type=jnp.float32)
    o_ref[...] = acc + b_ref[...]


def _tc_matmul(x2d, w, b, gath):
    n, d_in = x2d.shape
    tm = 1024
    return pl.pallas_call(
        _tc_matmul_body,
        grid=(n // tm,),
        in_specs=[
            pl.BlockSpec((tm, d_in), lambda i: (i, 0)),
            pl.BlockSpec((_PROJ, d_in), lambda i: (0, 0)),
            pl.BlockSpec((1, _PROJ), lambda i: (0, 0)),
            pl.BlockSpec(memory_space=pl.ANY),
        ],
        out_specs=pl.BlockSpec((tm, _PROJ), lambda i: (i, 0)),
        out_shape=jax.ShapeDtypeStruct((n, _OUT_D), jnp.float32),
        input_output_aliases={3: 0},
    )(x2d, w, b.reshape(1, _PROJ), gath)


def kernel(input_layer, tag_ids, pretag_ids, predicate_mask,
           predicate_embeddings, tag_embeddings, pretag_embeddings,
           linear_w, linear_b):
    b, s, d_in = input_layer.shape
    n = b * s
    x2d = input_layer.reshape(n, d_in)
    gath = _sc_gather(
        tag_ids.reshape(n).astype(jnp.int32),
        pretag_ids.reshape(n).astype(jnp.int32),
        predicate_mask.reshape(n).astype(jnp.int32),
        tag_embeddings, pretag_embeddings, predicate_embeddings,
    )
    out = _tc_matmul(x2d, linear_w, linear_b, gath)
    return out.reshape(b, s, _OUT_D)


# speedup vs baseline: 1.2737x; 1.0705x over previous
import functools

import jax
import jax.numpy as jnp
from jax import lax
from jax.experimental import pallas as pl
from jax.experimental.pallas import tpu as pltpu
from jax.experimental.pallas import tpu_sc as plsc

_NUM_CORES = 2
_NUM_SUBCORES = 16
_NW = _NUM_CORES * _NUM_SUBCORES

_EMB = 128
_PROJ = 384
_OUT_D = _PROJ + 3 * _EMB

_SB = 256
_NBUF = 3


def _sc_gather_body(tag_ids, pretag_ids, pred_ids,
                    tag_tab, pretag_tab, pred_tab,
                    out, idx_tag, idx_pred, idx_pretag,
                    pred_shared, bufs, gsems, wsems, *, chunk):
    wid = lax.axis_index("s") * _NUM_CORES + lax.axis_index("c")
    base = wid * chunk
    n_sb = chunk // _SB

    @pl.when(lax.axis_index("s") == 0)
    def _():
        pltpu.sync_copy(pred_tab, pred_shared)

    pltpu.sync_copy(tag_ids.at[pl.ds(base, chunk)], idx_tag)
    pltpu.sync_copy(pred_ids.at[pl.ds(base, chunk)], idx_pred)
    pltpu.sync_copy(pretag_ids.at[pl.ds(base, chunk)], idx_pretag)
    plsc.subcore_barrier()

    steps = []
    for idx_v, tab, col in (
        (idx_tag, tag_tab, _PROJ),
        (idx_pred, pred_shared, _PROJ + _EMB),
        (idx_pretag, pretag_tab, _PROJ + 2 * _EMB),
    ):
        for j in range(n_sb):
            steps.append((idx_v, tab, col, j))
    n_steps = len(steps)

    def start_gather(s):
        idx_v, tab, _, j = steps[s]
        b = s % _NBUF
        return pltpu.async_copy(
            tab.at[idx_v.at[pl.ds(j * _SB, _SB)]], bufs[b], gsems[b])

    def start_write(s):
        _, _, col, j = steps[s]
        b = s % _NBUF
        return pltpu.async_copy(
            bufs[b], out.at[pl.ds(base + j * _SB, _SB), pl.ds(col, _EMB)],
            wsems[b])

    ghandles, whandles = {}, {}
    for s in range(min(_NBUF - 1, n_steps)):
        ghandles[s] = start_gather(s)
    for s in range(n_steps):
        g = s + _NBUF - 1
        if g < n_steps:
            prev_w = g - _NBUF
            if prev_w >= 0:
                whandles.pop(prev_w).wait()
            ghandles[g] = start_gather(g)
        ghandles.pop(s).wait()
        whandles[s] = start_write(s)
    for s in sorted(whandles):
        whandles.pop(s).wait()


def _sc_gather(tag_ids, pretag_ids, pred_ids, tag_tab, pretag_tab, pred_tab):
    n = tag_ids.shape[0]
    chunk = n // _NW
    mesh = plsc.VectorSubcoreMesh(core_axis_name="c", subcore_axis_name="s")
    return pl.kernel(
        functools.partial(_sc_gather_body, chunk=chunk),
        out_type=jax.ShapeDtypeStruct((n, _OUT_D), jnp.float32),
        mesh=mesh,
        scratch_types=[
            pltpu.VMEM((chunk,), jnp.int32),
            pltpu.VMEM((chunk,), jnp.int32),
            pltpu.VMEM((chunk,), jnp.int32),
            pltpu.VMEM_SHARED((2, _EMB), jnp.float32),
            [pltpu.VMEM((_SB, _EMB), jnp.float32) for _ in range(_NBUF)],
            [pltpu.SemaphoreType.DMA for _ in range(_NBUF)],
            [pltpu.SemaphoreType.DMA for _ in range(_NBUF)],
        ],
    )(tag_ids, pretag_ids, pred_ids, tag_tab, pretag_tab, pred_tab)


def _tc_matmul_body(x_ref, w_ref, b_ref, gath_ref, o_ref):
    del gath_ref
    acc = lax.dot_general(x_ref[...].astype(jnp.bfloat16),
                          w_ref[...].astype(jnp.bfloat16),
                          (((1,), (1,)), ((), ())),
                          preferred_element_type=jnp.float32)
    o_ref[...] = acc + b_ref[...]


def _tc_matmul(x2d, w, b, gath):
    n, d_in = x2d.shape
    tm = 2048
    return pl.pallas_call(
        _tc_matmul_body,
        grid=(n // tm,),
        in_specs=[
            pl.BlockSpec((tm, d_in), lambda i: (i, 0)),
            pl.BlockSpec((_PROJ, d_in), lambda i: (0, 0)),
            pl.BlockSpec((1, _PROJ), lambda i: (0, 0)),
            pl.BlockSpec(memory_space=pl.ANY),
        ],
        out_specs=pl.BlockSpec((tm, _PROJ), lambda i: (i, 0)),
        out_shape=jax.ShapeDtypeStruct((n, _OUT_D), jnp.float32),
        input_output_aliases={3: 0},
    )(x2d, w, b.reshape(1, _PROJ), gath)


def kernel(input_layer, tag_ids, pretag_ids, predicate_mask,
           predicate_embeddings, tag_embeddings, pretag_embeddings,
           linear_w, linear_b):
    b, s, d_in = input_layer.shape
    n = b * s
    x2d = input_layer.reshape(n, d_in)
    gath = _sc_gather(
        tag_ids.reshape(n).astype(jnp.int32),
        pretag_ids.reshape(n).astype(jnp.int32),
        predicate_mask.reshape(n).astype(jnp.int32),
        tag_embeddings, pretag_embeddings, predicate_embeddings,
    )
    out = _tc_matmul(x2d, linear_w, linear_b, gath)
    return out.reshape(b, s, _OUT_D)


# speedup vs baseline: 1.3003x; 1.0209x over previous
import functools

import jax
import jax.numpy as jnp
from jax import lax
from jax.experimental import pallas as pl
from jax.experimental.pallas import tpu as pltpu
from jax.experimental.pallas import tpu_sc as plsc

_NUM_CORES = 2
_NUM_SUBCORES = 16
_NW = _NUM_CORES * _NUM_SUBCORES

_EMB = 128
_PROJ = 384
_OUT_D = _PROJ + 3 * _EMB

_SB = 256
_NBUF = 3


def _sc_gather_body(tag_ids, pretag_ids, pred_ids,
                    tag_tab, pretag_tab, pred_tab,
                    out, idx_tag, idx_pred, idx_pretag,
                    pred_shared, bufs, gsems, wsems, *, chunk):
    wid = lax.axis_index("s") * _NUM_CORES + lax.axis_index("c")
    base = wid * chunk
    n_sb = chunk // _SB

    @pl.when(lax.axis_index("s") == 0)
    def _():
        pltpu.sync_copy(pred_tab, pred_shared)

    pltpu.sync_copy(tag_ids.at[pl.ds(base, chunk)], idx_tag)
    pltpu.sync_copy(pred_ids.at[pl.ds(base, chunk)], idx_pred)
    pltpu.sync_copy(pretag_ids.at[pl.ds(base, chunk)], idx_pretag)
    plsc.subcore_barrier()

    steps = []
    for idx_v, tab, col in (
        (idx_tag, tag_tab, _PROJ),
        (idx_pred, pred_shared, _PROJ + _EMB),
        (idx_pretag, pretag_tab, _PROJ + 2 * _EMB),
    ):
        for j in range(n_sb):
            steps.append((idx_v, tab, col, j))
    n_steps = len(steps)

    def start_gather(s):
        idx_v, tab, _, j = steps[s]
        b = s % _NBUF
        return pltpu.async_copy(
            tab.at[idx_v.at[pl.ds(j * _SB, _SB)]], bufs[b], gsems[b])

    def start_write(s):
        _, _, col, j = steps[s]
        b = s % _NBUF
        return pltpu.async_copy(
            bufs[b], out.at[pl.ds(base + j * _SB, _SB), pl.ds(col, _EMB)],
            wsems[b])

    ghandles, whandles = {}, {}
    for s in range(min(_NBUF - 1, n_steps)):
        ghandles[s] = start_gather(s)
    for s in range(n_steps):
        g = s + _NBUF - 1
        if g < n_steps:
            prev_w = g - _NBUF
            if prev_w >= 0:
                whandles.pop(prev_w).wait()
            ghandles[g] = start_gather(g)
        ghandles.pop(s).wait()
        whandles[s] = start_write(s)
    for s in sorted(whandles):
        whandles.pop(s).wait()


def _sc_gather(tag_ids, pretag_ids, pred_ids, tag_tab, pretag_tab, pred_tab):
    n = tag_ids.shape[0]
    chunk = n // _NW
    mesh = plsc.VectorSubcoreMesh(core_axis_name="c", subcore_axis_name="s")
    return pl.kernel(
        functools.partial(_sc_gather_body, chunk=chunk),
        out_type=jax.ShapeDtypeStruct((n, _OUT_D), jnp.float32),
        mesh=mesh,
        scratch_types=[
            pltpu.VMEM((chunk,), jnp.int32),
            pltpu.VMEM((chunk,), jnp.int32),
            pltpu.VMEM((chunk,), jnp.int32),
            pltpu.VMEM_SHARED((2, _EMB), jnp.float32),
            [pltpu.VMEM((_SB, _EMB), jnp.float32) for _ in range(_NBUF)],
            [pltpu.SemaphoreType.DMA for _ in range(_NBUF)],
            [pltpu.SemaphoreType.DMA for _ in range(_NBUF)],
        ],
    )(tag_ids, pretag_ids, pred_ids, tag_tab, pretag_tab, pred_tab)


def _tc_matmul_body(x_ref, w_ref, b_ref, gath_ref, o_ref):
    del gath_ref
    acc = lax.dot_general(x_ref[...].astype(jnp.bfloat16),
                          w_ref[...].astype(jnp.bfloat16),
                          (((1,), (1,)), ((), ())),
                          preferred_element_type=jnp.float32)
    o_ref[...] = acc + b_ref[...]


def _tc_matmul(x2d, w, b, gath):
    n, d_in = x2d.shape
    tm = 4096
    return pl.pallas_call(
        _tc_matmul_body,
        grid=(n // tm,),
        in_specs=[
            pl.BlockSpec((tm, d_in), lambda i: (i, 0)),
            pl.BlockSpec((_PROJ, d_in), lambda i: (0, 0)),
            pl.BlockSpec((1, _PROJ), lambda i: (0, 0)),
            pl.BlockSpec(memory_space=pl.ANY),
        ],
        out_specs=pl.BlockSpec((tm, _PROJ), lambda i: (i, 0)),
        out_shape=jax.ShapeDtypeStruct((n, _OUT_D), jnp.float32),
        input_output_aliases={3: 0},
    )(x2d, w, b.reshape(1, _PROJ), gath)


def kernel(input_layer, tag_ids, pretag_ids, predicate_mask,
           predicate_embeddings, tag_embeddings, pretag_embeddings,
           linear_w, linear_b):
    b, s, d_in = input_layer.shape
    n = b * s
    x2d = input_layer.reshape(n, d_in)
    gath = _sc_gather(
        tag_ids.reshape(n).astype(jnp.int32),
        pretag_ids.reshape(n).astype(jnp.int32),
        predicate_mask.reshape(n).astype(jnp.int32),
        tag_embeddings, pretag_embeddings, predicate_embeddings,
    )
    out = _tc_matmul(x2d, linear_w, linear_b, gath)
    return out.reshape(b, s, _OUT_D)


# speedup vs baseline: 1.3165x; 1.0124x over previous
import functools

import jax
import jax.numpy as jnp
from jax import lax
from jax.experimental import pallas as pl
from jax.experimental.pallas import tpu as pltpu
from jax.experimental.pallas import tpu_sc as plsc

_NUM_CORES = 2
_NUM_SUBCORES = 16
_NW = _NUM_CORES * _NUM_SUBCORES

_EMB = 128
_PROJ = 384
_OUT_D = _PROJ + 3 * _EMB

_SB = 256
_NBUF = 3


def _sc_gather_body(tag_ids, pretag_ids, pred_ids,
                    tag_tab, pretag_tab, pred_tab,
                    out, idx_tag, idx_pred, idx_pretag,
                    pred_shared, bufs, gsems, wsems, isem, *, chunk):
    wid = lax.axis_index("s") * _NUM_CORES + lax.axis_index("c")
    base = wid * chunk
    n_sb = chunk // _SB

    h_tag = pltpu.async_copy(tag_ids.at[pl.ds(base, chunk)], idx_tag, isem)
    h_pred = pltpu.async_copy(pred_ids.at[pl.ds(base, chunk)], idx_pred, isem)
    h_pretag = pltpu.async_copy(pretag_ids.at[pl.ds(base, chunk)], idx_pretag,
                                isem)

    @pl.when(lax.axis_index("s") == 0)
    def _():
        pltpu.sync_copy(pred_tab, pred_shared)

    plsc.subcore_barrier()
    h_tag.wait()
    h_pred.wait()
    h_pretag.wait()

    steps = []
    for idx_v, tab, col in (
        (idx_tag, tag_tab, _PROJ),
        (idx_pred, pred_shared, _PROJ + _EMB),
        (idx_pretag, pretag_tab, _PROJ + 2 * _EMB),
    ):
        for j in range(n_sb):
            steps.append((idx_v, tab, col, j))
    n_steps = len(steps)

    def start_gather(s):
        idx_v, tab, _, j = steps[s]
        b = s % _NBUF
        return pltpu.async_copy(
            tab.at[idx_v.at[pl.ds(j * _SB, _SB)]], bufs[b], gsems[b])

    def start_write(s):
        _, _, col, j = steps[s]
        b = s % _NBUF
        return pltpu.async_copy(
            bufs[b], out.at[pl.ds(base + j * _SB, _SB), pl.ds(col, _EMB)],
            wsems[b])

    ghandles, whandles = {}, {}
    for s in range(min(_NBUF - 1, n_steps)):
        ghandles[s] = start_gather(s)
    for s in range(n_steps):
        g = s + _NBUF - 1
        if g < n_steps:
            prev_w = g - _NBUF
            if prev_w >= 0:
                whandles.pop(prev_w).wait()
            ghandles[g] = start_gather(g)
        ghandles.pop(s).wait()
        whandles[s] = start_write(s)
    for s in sorted(whandles):
        whandles.pop(s).wait()


def _sc_gather(tag_ids, pretag_ids, pred_ids, tag_tab, pretag_tab, pred_tab):
    n = tag_ids.shape[0]
    chunk = n // _NW
    mesh = plsc.VectorSubcoreMesh(core_axis_name="c", subcore_axis_name="s")
    return pl.kernel(
        functools.partial(_sc_gather_body, chunk=chunk),
        out_type=jax.ShapeDtypeStruct((n, _OUT_D), jnp.float32),
        mesh=mesh,
        scratch_types=[
            pltpu.VMEM((chunk,), jnp.int32),
            pltpu.VMEM((chunk,), jnp.int32),
            pltpu.VMEM((chunk,), jnp.int32),
            pltpu.VMEM_SHARED((2, _EMB), jnp.float32),
            [pltpu.VMEM((_SB, _EMB), jnp.float32) for _ in range(_NBUF)],
            [pltpu.SemaphoreType.DMA for _ in range(_NBUF)],
            [pltpu.SemaphoreType.DMA for _ in range(_NBUF)],
            pltpu.SemaphoreType.DMA,
        ],
    )(tag_ids, pretag_ids, pred_ids, tag_tab, pretag_tab, pred_tab)


def _tc_matmul_body(x_ref, w_ref, b_ref, gath_ref, o_ref):
    del gath_ref
    acc = lax.dot_general(x_ref[...].astype(jnp.bfloat16),
                          w_ref[...].astype(jnp.bfloat16),
                          (((1,), (1,)), ((), ())),
                          preferred_element_type=jnp.float32)
    o_ref[...] = acc + b_ref[...]


def _tc_matmul(x2d, w, b, gath):
    n, d_in = x2d.shape
    tm = 4096
    return pl.pallas_call(
        _tc_matmul_body,
        grid=(n // tm,),
        in_specs=[
            pl.BlockSpec((tm, d_in), lambda i: (i, 0)),
            pl.BlockSpec((_PROJ, d_in), lambda i: (0, 0)),
            pl.BlockSpec((1, _PROJ), lambda i: (0, 0)),
            pl.BlockSpec(memory_space=pl.ANY),
        ],
        out_specs=pl.BlockSpec((tm, _PROJ), lambda i: (i, 0)),
        out_shape=jax.ShapeDtypeStruct((n, _OUT_D), jnp.float32),
        input_output_aliases={3: 0},
    )(x2d, w, b.reshape(1, _PROJ), gath)


def kernel(input_layer, tag_ids, pretag_ids, predicate_mask,
           predicate_embeddings, tag_embeddings, pretag_embeddings,
           linear_w, linear_b):
    b, s, d_in = input_layer.shape
    n = b * s
    x2d = input_layer.reshape(n, d_in)
    gath = _sc_gather(
        tag_ids.reshape(n).astype(jnp.int32),
        pretag_ids.reshape(n).astype(jnp.int32),
        predicate_mask.reshape(n).astype(jnp.int32),
        tag_embeddings, pretag_embeddings, predicate_embeddings,
    )
    out = _tc_matmul(x2d, linear_w, linear_b, gath)
    return out.reshape(b, s, _OUT_D)


# speedup vs baseline: 1.3212x; 1.0036x over previous
import functools

import jax
import jax.numpy as jnp
from jax import lax
from jax.experimental import pallas as pl
from jax.experimental.pallas import tpu as pltpu
from jax.experimental.pallas import tpu_sc as plsc

_NUM_CORES = 2
_NUM_SUBCORES = 16
_NW = _NUM_CORES * _NUM_SUBCORES

_EMB = 128
_PROJ = 384
_OUT_D = _PROJ + 3 * _EMB

_SB = 128
_NBUF = 5


def _sc_gather_body(tag_ids, pretag_ids, pred_ids,
                    tag_tab, pretag_tab, pred_tab,
                    out, idx_tag, idx_pred, idx_pretag,
                    pred_shared, bufs, gsems, wsems, isem, *, chunk):
    wid = lax.axis_index("s") * _NUM_CORES + lax.axis_index("c")
    base = wid * chunk
    n_sb = chunk // _SB

    h_tag = pltpu.async_copy(tag_ids.at[pl.ds(base, chunk)], idx_tag, isem)
    h_pred = pltpu.async_copy(pred_ids.at[pl.ds(base, chunk)], idx_pred, isem)
    h_pretag = pltpu.async_copy(pretag_ids.at[pl.ds(base, chunk)], idx_pretag,
                                isem)

    @pl.when(lax.axis_index("s") == 0)
    def _():
        pltpu.sync_copy(pred_tab, pred_shared)

    plsc.subcore_barrier()
    h_tag.wait()
    h_pred.wait()
    h_pretag.wait()

    steps = []
    for idx_v, tab, col in (
        (idx_tag, tag_tab, _PROJ),
        (idx_pred, pred_shared, _PROJ + _EMB),
        (idx_pretag, pretag_tab, _PROJ + 2 * _EMB),
    ):
        for j in range(n_sb):
            steps.append((idx_v, tab, col, j))
    n_steps = len(steps)

    def start_gather(s):
        idx_v, tab, _, j = steps[s]
        b = s % _NBUF
        return pltpu.async_copy(
            tab.at[idx_v.at[pl.ds(j * _SB, _SB)]], bufs[b], gsems[b])

    def start_write(s):
        _, _, col, j = steps[s]
        b = s % _NBUF
        return pltpu.async_copy(
            bufs[b], out.at[pl.ds(base + j * _SB, _SB), pl.ds(col, _EMB)],
            wsems[b])

    ghandles, whandles = {}, {}
    for s in range(min(_NBUF - 1, n_steps)):
        ghandles[s] = start_gather(s)
    for s in range(n_steps):
        g = s + _NBUF - 1
        if g < n_steps:
            prev_w = g - _NBUF
            if prev_w >= 0:
                whandles.pop(prev_w).wait()
            ghandles[g] = start_gather(g)
        ghandles.pop(s).wait()
        whandles[s] = start_write(s)
    for s in sorted(whandles):
        whandles.pop(s).wait()


def _sc_gather(tag_ids, pretag_ids, pred_ids, tag_tab, pretag_tab, pred_tab):
    n = tag_ids.shape[0]
    chunk = n // _NW
    mesh = plsc.VectorSubcoreMesh(core_axis_name="c", subcore_axis_name="s")
    return pl.kernel(
        functools.partial(_sc_gather_body, chunk=chunk),
        out_type=jax.ShapeDtypeStruct((n, _OUT_D), jnp.float32),
        mesh=mesh,
        scratch_types=[
            pltpu.VMEM((chunk,), jnp.int32),
            pltpu.VMEM((chunk,), jnp.int32),
            pltpu.VMEM((chunk,), jnp.int32),
            pltpu.VMEM_SHARED((2, _EMB), jnp.float32),
            [pltpu.VMEM((_SB, _EMB), jnp.float32) for _ in range(_NBUF)],
            [pltpu.SemaphoreType.DMA for _ in range(_NBUF)],
            [pltpu.SemaphoreType.DMA for _ in range(_NBUF)],
            pltpu.SemaphoreType.DMA,
        ],
    )(tag_ids, pretag_ids, pred_ids, tag_tab, pretag_tab, pred_tab)


def _tc_matmul_body(x_ref, w_ref, b_ref, gath_ref, o_ref):
    del gath_ref
    acc = lax.dot_general(x_ref[...].astype(jnp.bfloat16),
                          w_ref[...].astype(jnp.bfloat16),
                          (((1,), (1,)), ((), ())),
                          preferred_element_type=jnp.float32)
    o_ref[...] = acc + b_ref[...]


def _tc_matmul(x2d, w, b, gath):
    n, d_in = x2d.shape
    tm = 4096
    return pl.pallas_call(
        _tc_matmul_body,
        grid=(n // tm,),
        in_specs=[
            pl.BlockSpec((tm, d_in), lambda i: (i, 0)),
            pl.BlockSpec((_PROJ, d_in), lambda i: (0, 0)),
            pl.BlockSpec((1, _PROJ), lambda i: (0, 0)),
            pl.BlockSpec(memory_space=pl.ANY),
        ],
        out_specs=pl.BlockSpec((tm, _PROJ), lambda i: (i, 0)),
        out_shape=jax.ShapeDtypeStruct((n, _OUT_D), jnp.float32),
        input_output_aliases={3: 0},
    )(x2d, w, b.reshape(1, _PROJ), gath)


def kernel(input_layer, tag_ids, pretag_ids, predicate_mask,
           predicate_embeddings, tag_embeddings, pretag_embeddings,
           linear_w, linear_b):
    b, s, d_in = input_layer.shape
    n = b * s
    x2d = input_layer.reshape(n, d_in)
    gath = _sc_gather(
        tag_ids.reshape(n).astype(jnp.int32),
        pretag_ids.reshape(n).astype(jnp.int32),
        predicate_mask.reshape(n).astype(jnp.int32),
        tag_embeddings, pretag_embeddings, predicate_embeddings,
    )
    out = _tc_matmul(x2d, linear_w, linear_b, gath)
    return out.reshape(b, s, _OUT_D)
